# Initial kernel scaffold; baseline (speedup 1.0000x reference)
import jax
import jax.numpy as jnp
from jax.experimental import pallas as pl


def kernel(v_coords, v_weights, edge_verts, edge_weights, tri_verts, tri_weights, dirs):
    raise NotImplementedError("write your pallas kernel here")


# speedup vs baseline: 15.4672x; 15.4672x over previous
import functools

import jax
import jax.numpy as jnp
from jax import lax
from jax.experimental import pallas as pl
from jax.experimental.pallas import tpu as pltpu
from jax.experimental.pallas import tpu_sc as plsc

D = 32
H = 256
DH = D * H

N_V = 100000
N_E = 500000
N_T = 500000

NW = 32
NC = 2

NVP = 102400
NEP = 524288
NTP = 524288

VPT = NVP // NW
EPT = NEP // NW
TPT = NTP // NW

VB = 128
EBLK = 512
TBLK = 1024
NVB = VPT // VB
NEB = EPT // EBLK
NTB = TPT // TBLK
ERPT = EPT * 2 // 128
TRPT = TPT * 3 // 128

VBLK = 2048
NGRID = NVP // VBLK


def _maxsq_body(x_ref, o_ref):
    i = pl.program_id(0)
    c = x_ref[...]
    s = jnp.sum(c * c, axis=1)
    m = jnp.max(s)

    @pl.when(i == 0)
    def _():
        o_ref[0, 0] = m

    @pl.when(i > 0)
    def _():
        o_ref[0, 0] = jnp.maximum(o_ref[0, 0], m)


def _quant_body(ms_ref, x_ref, d_ref, o_ref):
    m = jnp.sqrt(ms_ref[0, 0])
    c = x_ref[...]
    dt = d_ref[...]
    h = (c[:, 0:1] * dt[0:1, :]
         + c[:, 1:2] * dt[1:2, :]
         + c[:, 2:3] * dt[2:3, :])
    idx = jnp.ceil((jnp.float32(H - 1) * (m + h)) / (jnp.float32(2.0) * m))
    idx = jnp.clip(idx, 0.0, jnp.float32(H - 1))
    o_ref[...] = idx.astype(jnp.int32)


def _fin_body(h_ref, o_ref):
    x = h_ref[...]
    j = lax.broadcasted_iota(jnp.int32, (D, NW * D), 1)
    dd = lax.broadcasted_iota(jnp.int32, (D, NW * D), 0)
    sel = ((j & (D - 1)) == dd).astype(jnp.float32)
    acc = jnp.dot(sel, x, preferred_element_type=jnp.float32)
    s_i = lax.broadcasted_iota(jnp.int32, (H, H), 0)
    t_i = lax.broadcasted_iota(jnp.int32, (H, H), 1)
    tri = (s_i <= t_i).astype(jnp.float32)
    o_ref[...] = jnp.dot(acc, tri, preferred_element_type=jnp.float32)


def _sc_hist_body(vidx, vw, ev, ew, tv, tw, out,
                  hist, e_idx, e_rows, e_w, t_idx, t_rows, t_w,
                  v_rows, v_w, sem):
    wid = lax.axis_index("s") * NC + lax.axis_index("c")

    offs_lo = lax.iota(jnp.int32, 16) * H
    offs_hi = offs_lo + 16 * H
    zeros16 = jnp.zeros((16,), jnp.float32)

    def zi(i, _):
        hist[pl.ds(i * 16, 16)] = zeros16
        return 0

    lax.fori_loop(0, DH // 16, zi, 0)

    vbase = wid * VPT

    def vblk(b, carry):
        base = vbase + b * VB
        pltpu.sync_copy(vidx.at[pl.ds(base, VB)], v_rows)
        pltpu.sync_copy(vw.at[pl.ds(base, VB)], v_w)

        def vbody(g, c2):
            wv16 = v_w[pl.ds(g * 16, 16)]
            for j in range(16):
                i = g * 16 + j
                wv = jnp.full((16,), wv16[j], jnp.float32)
                r0 = v_rows[i, pl.ds(0, 16)]
                r1 = v_rows[i, pl.ds(16, 16)]
                plsc.addupdate_scatter(hist, [r0 + offs_lo], wv)
                plsc.addupdate_scatter(hist, [r1 + offs_hi], wv)
            return c2

        return lax.fori_loop(0, VB // 16, vbody, carry)

    lax.fori_loop(0, NVB, vblk, 0)

    ebase_rows = wid * ERPT

    def eblk(b, carry):
        rb = ebase_rows + b * 8
        pltpu.sync_copy(ev.at[pl.ds(rb, 8)], e_idx)
        pltpu.sync_copy(ew.at[pl.ds(wid * EPT + b * EBLK, EBLK)], e_w)

        def esub(s, c1_):
            d1 = pltpu.async_copy(vidx.at[e_idx.at[2 * s]],
                                  e_rows.at[pl.ds(0, 128)], sem)
            d2 = pltpu.async_copy(vidx.at[e_idx.at[2 * s + 1]],
                                  e_rows.at[pl.ds(128, 128)], sem)
            d1.wait()
            d2.wait()

            def ebody(g, c2_):
                wv16 = e_w[pl.ds(s * 128 + g * 16, 16)]
                for j in range(16):
                    i = g * 16 + j
                    wv = jnp.full((16,), wv16[j], jnp.float32)
                    a0 = e_rows[2 * i, pl.ds(0, 16)]
                    a1 = e_rows[2 * i, pl.ds(16, 16)]
                    b0 = e_rows[2 * i + 1, pl.ds(0, 16)]
                    b1 = e_rows[2 * i + 1, pl.ds(16, 16)]
                    m0 = jnp.maximum(a0, b0) + offs_lo
                    m1 = jnp.maximum(a1, b1) + offs_hi
                    plsc.addupdate_scatter(hist, [m0], wv)
                    plsc.addupdate_scatter(hist, [m1], wv)
                return c2_

            return lax.fori_loop(0, 8, ebody, c1_)

        return lax.fori_loop(0, EBLK // 128, esub, carry)

    lax.fori_loop(0, NEB, eblk, 0)

    tbase_rows = wid * TRPT

    def tblk(b, carry):
        rb = tbase_rows + b * 24
        pltpu.sync_copy(tv.at[pl.ds(rb, 24)], t_idx)
        pltpu.sync_copy(tw.at[pl.ds(wid * TPT + b * TBLK, TBLK)], t_w)

        def tsub(s, c1x):
            d1 = pltpu.async_copy(vidx.at[t_idx.at[3 * s]],
                                  t_rows.at[pl.ds(0, 128)], sem)
            d2 = pltpu.async_copy(vidx.at[t_idx.at[3 * s + 1]],
                                  t_rows.at[pl.ds(128, 128)], sem)
            d3 = pltpu.async_copy(vidx.at[t_idx.at[3 * s + 2]],
                                  t_rows.at[pl.ds(256, 128)], sem)
            d1.wait()
            d2.wait()
            d3.wait()

            def tbody(g, c2_):
                wv16 = -t_w[pl.ds(s * 128 + g * 16, 16)]
                for j in range(16):
                    i = g * 16 + j
                    wv = jnp.full((16,), wv16[j], jnp.float32)
                    a0 = t_rows[3 * i, pl.ds(0, 16)]
                    a1 = t_rows[3 * i, pl.ds(16, 16)]
                    b0 = t_rows[3 * i + 1, pl.ds(0, 16)]
                    b1 = t_rows[3 * i + 1, pl.ds(16, 16)]
                    c0 = t_rows[3 * i + 2, pl.ds(0, 16)]
                    c1_ = t_rows[3 * i + 2, pl.ds(16, 16)]
                    m0 = jnp.maximum(jnp.maximum(a0, b0), c0) + offs_lo
                    m1 = jnp.maximum(jnp.maximum(a1, b1), c1_) + offs_hi
                    plsc.addupdate_scatter(hist, [m0], wv)
                    plsc.addupdate_scatter(hist, [m1], wv)
                return c2_

            return lax.fori_loop(0, 8, tbody, c1x)

        return lax.fori_loop(0, TBLK // 128, tsub, carry)

    lax.fori_loop(0, NTB, tblk, 0)

    pltpu.sync_copy(hist, out.at[wid])


def _make_sc_hist():
    mesh = plsc.VectorSubcoreMesh(core_axis_name="c", subcore_axis_name="s")
    return functools.partial(
        pl.kernel,
        mesh=mesh,
        compiler_params=pltpu.CompilerParams(
            needs_layout_passes=False, use_tc_tiling_on_sc=False),
        out_type=jax.ShapeDtypeStruct((NW, DH), jnp.float32),
        scratch_types=[
            pltpu.VMEM((DH,), jnp.float32),
            pltpu.VMEM((8, 128), jnp.int32),
            pltpu.VMEM((256, D), jnp.int32),
            pltpu.VMEM((EBLK,), jnp.float32),
            pltpu.VMEM((24, 128), jnp.int32),
            pltpu.VMEM((384, D), jnp.int32),
            pltpu.VMEM((TBLK,), jnp.float32),
            pltpu.VMEM((VB, D), jnp.int32),
            pltpu.VMEM((VB,), jnp.float32),
            pltpu.SemaphoreType.DMA,
        ],
    )(_sc_hist_body)


_sc_hist = _make_sc_hist()


def kernel(v_coords, v_weights, edge_verts, edge_weights, tri_verts,
           tri_weights, dirs):
    vc = jnp.pad(v_coords, ((0, NVP - N_V), (0, 0)))
    vwp = jnp.pad(v_weights, (0, NVP - N_V))
    evf = jnp.pad(edge_verts.astype(jnp.int32),
                  ((0, NEP - N_E), (0, 0))).reshape(NEP * 2 // 128, 128)
    ewp = jnp.pad(edge_weights, (0, NEP - N_E))
    tvf = jnp.pad(tri_verts.astype(jnp.int32),
                  ((0, NTP - N_T), (0, 0))).reshape(NTP * 3 // 128, 128)
    twp = jnp.pad(tri_weights, (0, NTP - N_T))
    dirs_t = dirs.T

    maxsq = pl.pallas_call(
        _maxsq_body,
        grid=(NGRID,),
        in_specs=[pl.BlockSpec((VBLK, 3), lambda i: (i, 0))],
        out_specs=pl.BlockSpec(memory_space=pltpu.SMEM),
        out_shape=jax.ShapeDtypeStruct((1, 1), jnp.float32),
    )(vc)

    vidx = pl.pallas_call(
        _quant_body,
        grid=(NGRID,),
        in_specs=[
            pl.BlockSpec(memory_space=pltpu.SMEM),
            pl.BlockSpec((VBLK, 3), lambda i: (i, 0)),
            pl.BlockSpec((3, D), lambda i: (0, 0)),
        ],
        out_specs=pl.BlockSpec((VBLK, D), lambda i: (i, 0)),
        out_shape=jax.ShapeDtypeStruct((NVP, D), jnp.int32),
    )(maxsq, vc, dirs_t)

    hists = _sc_hist(vidx, vwp, evf, ewp, tvf, twp)

    out = pl.pallas_call(
        _fin_body,
        in_specs=[pl.BlockSpec((NW * D, H), lambda: (0, 0))],
        out_specs=pl.BlockSpec((D, H), lambda: (0, 0)),
        out_shape=jax.ShapeDtypeStruct((D, H), jnp.float32),
    )(hists.reshape(NW * D, H))

    return out


# speedup vs baseline: 17.2405x; 1.1146x over previous
import functools

import jax
import jax.numpy as jnp
from jax import lax
from jax.experimental import pallas as pl
from jax.experimental.pallas import tpu as pltpu
from jax.experimental.pallas import tpu_sc as plsc

D = 32
H = 256
DH = D * H

N_V = 100000
N_E = 500000
N_T = 500000

NW = 32
NC = 2

NVP = 102400
NEP = 524288
NTP = 524288

VPT = NVP // NW
EPT = NEP // NW
TPT = NTP // NW

SB = 2048
UN = 512
NUS = SB // UN
NSE = EPT // SB
NST = TPT // SB

VB = 128
NVB = VPT // VB

VBLK = 2048
NGRID = NVP // VBLK


def _maxsq_body(x_ref, o_ref):
    i = pl.program_id(0)
    c = x_ref[...]
    s = jnp.sum(c * c, axis=1)
    m = jnp.max(s)

    @pl.when(i == 0)
    def _():
        o_ref[0, 0] = m

    @pl.when(i > 0)
    def _():
        o_ref[0, 0] = jnp.maximum(o_ref[0, 0], m)


def _quant_body(ms_ref, x_ref, d_ref, o_ref):
    m = jnp.sqrt(ms_ref[0, 0])
    c = x_ref[...]
    dm = d_ref[...]
    h = (c[:, 0:1] * dm[0:1, :]
         + c[:, 1:2] * dm[1:2, :]
         + c[:, 2:3] * dm[2:3, :])
    idx = jnp.ceil((jnp.float32(H - 1) * (m + h)) / (jnp.float32(2.0) * m))
    idx = jnp.clip(idx, 0.0, jnp.float32(H - 1))
    o_ref[...] = (idx.astype(jnp.int32) - 128).astype(jnp.int8)


def _fin_body(h_ref, o_ref):
    x = h_ref[...]
    j = lax.broadcasted_iota(jnp.int32, (D, NW * D), 1)
    dd = lax.broadcasted_iota(jnp.int32, (D, NW * D), 0)
    sel = ((j & (D - 1)) == dd).astype(jnp.float32)
    acc = jnp.dot(sel, x, preferred_element_type=jnp.float32)
    s_i = lax.broadcasted_iota(jnp.int32, (H, H), 0)
    t_i = lax.broadcasted_iota(jnp.int32, (H, H), 1)
    tri = (s_i <= t_i).astype(jnp.float32)
    o_ref[...] = jnp.dot(acc, tri, preferred_element_type=jnp.float32)


def _unpack_idx(m8, offs_lo, offs_hi):
    a16, b16 = plsc.unpack(m8, format=plsc.PackFormat.INTERLEAVED,
                           preferred_element_type=jnp.int16)
    lo32, _ = plsc.unpack(a16, format=plsc.PackFormat.INTERLEAVED)
    hi32, _ = plsc.unpack(b16, format=plsc.PackFormat.INTERLEAVED)
    return lo32 + offs_lo, hi32 + offs_hi


def _sc_hist_body(vidx, vw, ev, ew, tv, tw, out,
                  hist, idx_v, rows, w_v, sem):
    wid = lax.axis_index("s") * NC + lax.axis_index("c")

    offs_lo = lax.iota(jnp.int32, 16) * H + 128
    offs_hi = offs_lo + 16 * H
    zeros16 = jnp.zeros((16,), jnp.float32)

    def zi(i, _):
        hist[pl.ds(i * 16, 16)] = zeros16
        return 0

    lax.fori_loop(0, DH // 16, zi, 0)

    vbase = wid * VPT

    def vblk(b, carry):
        base = vbase + b * VB
        pltpu.sync_copy(vidx.at[pl.ds(base, VB)], rows.at[0, pl.ds(0, VB)])
        pltpu.sync_copy(vw.at[pl.ds(base, VB)], w_v.at[pl.ds(0, VB)])

        def vbody(g, c2):
            wv16 = w_v[pl.ds(g * 16, 16)]
            for j in range(16):
                i = g * 16 + j
                wv = jnp.full((16,), wv16[j], jnp.float32)
                r8 = rows[0, i, pl.ds(0, 64)]
                f_lo, f_hi = _unpack_idx(r8, offs_lo, offs_hi)
                plsc.addupdate_scatter(hist, [f_lo], wv)
                plsc.addupdate_scatter(hist, [f_hi], wv)
            return c2

        return lax.fori_loop(0, VB // 16, vbody, carry)

    lax.fori_loop(0, NVB, vblk, 0)

    def simplex_pass(ids_hbm, w_hbm, n_super, rows_per, negate):
        ids_per_tile = (EPT if rows_per == 2 else TPT) * rows_per
        ids_per_super = SB * rows_per
        ids_per_unit = UN * rows_per
        n_gath = ids_per_unit // 128
        idbase = wid * ids_per_tile
        wbase = wid * (EPT if rows_per == 2 else TPT)

        def stage(k):
            pltpu.sync_copy(ids_hbm.at[pl.ds(idbase + k * ids_per_super,
                                             ids_per_super)],
                            idx_v.at[pl.ds(0, ids_per_super)])
            pltpu.sync_copy(w_hbm.at[pl.ds(wbase + k * SB, SB)],
                            w_v.at[pl.ds(0, SB)])

        def fire(un, p):
            for s in range(n_gath):
                pltpu.async_copy(
                    vidx.at[idx_v.at[pl.ds(un * ids_per_unit + s * 128, 128)]],
                    rows.at[p, pl.ds(s * 128, 128)],
                    sem.at[p])

        def wait(p):
            pltpu.make_async_copy(
                vidx.at[pl.ds(0, ids_per_unit)],
                rows.at[p, pl.ds(0, ids_per_unit)],
                sem.at[p]).wait()

        def compute(un, p):
            def body(g, c2):
                wv16 = w_v[pl.ds(un * UN + g * 16, 16)]
                if negate:
                    wv16 = -wv16
                for j in range(16):
                    i = g * 16 + j
                    wv = jnp.full((16,), wv16[j], jnp.float32)
                    r = rows_per * i
                    m8 = jnp.maximum(rows[p, r, pl.ds(0, 64)],
                                     rows[p, r + 1, pl.ds(0, 64)])
                    if rows_per == 3:
                        m8 = jnp.maximum(m8, rows[p, r + 2, pl.ds(0, 64)])
                    f_lo, f_hi = _unpack_idx(m8, offs_lo, offs_hi)
                    plsc.addupdate_scatter(hist, [f_lo], wv)
                    plsc.addupdate_scatter(hist, [f_hi], wv)
                return c2

            lax.fori_loop(0, UN // 16, body, 0)

        def super_body(k, carry):
            stage(k)
            fire(0, 0)
            fire(1, 1)
            for un in range(NUS):
                p = un & 1
                wait(p)
                compute(un, p)
                if un + 2 < NUS:
                    fire(un + 2, p)
            return carry

        lax.fori_loop(0, n_super, super_body, 0)

    simplex_pass(ev, ew, NSE, 2, False)
    simplex_pass(tv, tw, NST, 3, True)

    pltpu.sync_copy(hist, out.at[pl.ds(wid * DH, DH)])


def _make_sc_hist():
    mesh = plsc.VectorSubcoreMesh(core_axis_name="c", subcore_axis_name="s")
    return functools.partial(
        pl.kernel,
        mesh=mesh,
        compiler_params=pltpu.CompilerParams(
            needs_layout_passes=False, use_tc_tiling_on_sc=False),
        out_type=jax.ShapeDtypeStruct((NW * DH,), jnp.float32),
        scratch_types=[
            pltpu.VMEM((DH,), jnp.float32),
            pltpu.VMEM((SB * 3,), jnp.int32),
            pltpu.VMEM((2, UN * 3, 128), jnp.int8),
            pltpu.VMEM((SB,), jnp.float32),
            pltpu.SemaphoreType.DMA((2,)),
        ],
    )(_sc_hist_body)


_sc_hist = _make_sc_hist()


def kernel(v_coords, v_weights, edge_verts, edge_weights, tri_verts,
           tri_weights, dirs):
    vc = jnp.pad(v_coords, ((0, NVP - N_V), (0, 0)))
    vwp = jnp.pad(v_weights, (0, NVP - N_V))
    evf = jnp.pad(edge_verts.astype(jnp.int32).reshape(-1),
                  (0, (NEP - N_E) * 2))
    ewp = jnp.pad(edge_weights, (0, NEP - N_E))
    tvf = jnp.pad(tri_verts.astype(jnp.int32).reshape(-1),
                  (0, (NTP - N_T) * 3))
    twp = jnp.pad(tri_weights, (0, NTP - N_T))

    dirs_t = dirs.T
    dmat = jnp.zeros((3, 128), jnp.float32)
    cols = jnp.arange(16) * 4
    dmat = dmat.at[:, cols].set(dirs_t[:, :16])
    dmat = dmat.at[:, cols + 1].set(dirs_t[:, 16:])

    maxsq = pl.pallas_call(
        _maxsq_body,
        grid=(NGRID,),
        in_specs=[pl.BlockSpec((VBLK, 3), lambda i: (i, 0))],
        out_specs=pl.BlockSpec(memory_space=pltpu.SMEM),
        out_shape=jax.ShapeDtypeStruct((1, 1), jnp.float32),
    )(vc)

    vidx = pl.pallas_call(
        _quant_body,
        grid=(NGRID,),
        in_specs=[
            pl.BlockSpec(memory_space=pltpu.SMEM),
            pl.BlockSpec((VBLK, 3), lambda i: (i, 0)),
            pl.BlockSpec((3, 128), lambda i: (0, 0)),
        ],
        out_specs=pl.BlockSpec((VBLK, 128), lambda i: (i, 0)),
        out_shape=jax.ShapeDtypeStruct((NVP, 128), jnp.int8),
    )(maxsq, vc, dmat)

    hists = _sc_hist(vidx, vwp, evf, ewp, tvf, twp)

    out = pl.pallas_call(
        _fin_body,
        in_specs=[pl.BlockSpec((NW * D, H), lambda: (0, 0))],
        out_specs=pl.BlockSpec((D, H), lambda: (0, 0)),
        out_shape=jax.ShapeDtypeStruct((D, H), jnp.float32),
    )(hists.reshape(NW * D, H))

    return out


# speedup vs baseline: 27.8376x; 1.6147x over previous
import functools

import jax
import jax.numpy as jnp
from jax import lax
from jax.experimental import pallas as pl
from jax.experimental.pallas import tpu as pltpu
from jax.experimental.pallas import tpu_sc as plsc

D = 32
H = 256
DH = D * H

N_V = 100000
N_E = 500000
N_T = 500000

NW = 32
NC = 2

NVP = 102400
NEP = 524288
NTP = 524288

VPT = NVP // NW
EPT = NEP // NW
TPT = NTP // NW

SB = 2048
UN = 512
NUS = SB // UN
NSE = EPT // SB
NST = TPT // SB

VB = 128
NVB = VPT // VB

VBLK = 2048
NGRID = NVP // VBLK


def _maxsq_body(x_ref, o_ref):
    i = pl.program_id(0)
    c = x_ref[...]
    s = jnp.sum(c * c, axis=1)
    m = jnp.max(s)

    @pl.when(i == 0)
    def _():
        o_ref[0, 0] = m

    @pl.when(i > 0)
    def _():
        o_ref[0, 0] = jnp.maximum(o_ref[0, 0], m)


def _quant_body(ms_ref, x_ref, d_ref, o_ref):
    m = jnp.sqrt(ms_ref[0, 0])
    c = x_ref[...]
    dm = d_ref[...]
    h = (c[:, 0:1] * dm[0:1, :]
         + c[:, 1:2] * dm[1:2, :]
         + c[:, 2:3] * dm[2:3, :])
    idx = jnp.ceil((jnp.float32(H - 1) * (m + h)) / (jnp.float32(2.0) * m))
    idx = jnp.clip(idx, 0.0, jnp.float32(H - 1))
    o_ref[...] = (idx.astype(jnp.int32) - 128).astype(jnp.int8)


def _fin_body(h_ref, o_ref):
    x = h_ref[...]
    j = lax.broadcasted_iota(jnp.int32, (D, NW * D), 1)
    dd = lax.broadcasted_iota(jnp.int32, (D, NW * D), 0)
    sel = ((j & (D - 1)) == dd).astype(jnp.float32)
    acc = jnp.dot(sel, x, preferred_element_type=jnp.float32)
    s_i = lax.broadcasted_iota(jnp.int32, (H, H), 0)
    t_i = lax.broadcasted_iota(jnp.int32, (H, H), 1)
    tri = (s_i <= t_i).astype(jnp.float32)
    o_ref[...] = jnp.dot(acc, tri, preferred_element_type=jnp.float32)


def _unpack_idx(m8, offs_lo, offs_hi):
    a16, b16 = plsc.unpack(m8, format=plsc.PackFormat.INTERLEAVED,
                           preferred_element_type=jnp.int16)
    lo32, _ = plsc.unpack(a16, format=plsc.PackFormat.INTERLEAVED)
    hi32, _ = plsc.unpack(b16, format=plsc.PackFormat.INTERLEAVED)
    return lo32 + offs_lo, hi32 + offs_hi


def _sc_hist_body(vidx, vw, ev0, ev1, ew, tv0, tv1, tv2, tw, out,
                  hist, idx_v, rows, w_v, sem):
    wid = lax.axis_index("s") * NC + lax.axis_index("c")

    offs_lo = lax.iota(jnp.int32, 16) * H + 128
    offs_hi = offs_lo + 16 * H
    zeros16 = jnp.zeros((16,), jnp.float32)

    def zi(i, _):
        hist[pl.ds(i * 16, 16)] = zeros16
        return 0

    lax.fori_loop(0, DH // 16, zi, 0)

    vbase = wid * VPT

    def vblk(b, carry):
        base = vbase + b * VB
        pltpu.sync_copy(vidx.at[pl.ds(base, VB)], rows.at[0, pl.ds(0, VB)])
        pltpu.sync_copy(vw.at[pl.ds(base, VB)], w_v.at[pl.ds(0, VB)])

        def vbody(g, c2):
            wv16 = w_v[pl.ds(g * 16, 16)]
            for j in range(16):
                i = g * 16 + j
                wv = jnp.full((16,), wv16[j], jnp.float32)
                r8 = rows[0, i, pl.ds(0, 64)]
                f_lo, f_hi = _unpack_idx(r8, offs_lo, offs_hi)
                plsc.addupdate_scatter(hist, [f_lo], wv)
                plsc.addupdate_scatter(hist, [f_hi], wv)
            return c2

        return lax.fori_loop(0, VB // 16, vbody, carry)

    lax.fori_loop(0, NVB, vblk, 0)

    def simplex_pass(cols, w_hbm, n_super, rows_per, negate):
        ids_per_unit = UN * rows_per
        base = wid * (EPT if rows_per == 2 else TPT)

        def stage(k):
            for r in range(rows_per):
                pltpu.sync_copy(cols[r].at[pl.ds(base + k * SB, SB)],
                                idx_v.at[pl.ds(r * SB, SB)])
            pltpu.sync_copy(w_hbm.at[pl.ds(base + k * SB, SB)],
                            w_v.at[pl.ds(0, SB)])

        def fire(un, p):
            for r in range(rows_per):
                for s in range(UN // 128):
                    pltpu.async_copy(
                        vidx.at[idx_v.at[pl.ds(r * SB + un * UN + s * 128,
                                               128)]],
                        rows.at[p, pl.ds(r * UN + s * 128, 128)],
                        sem.at[p])

        def wait(p):
            pltpu.make_async_copy(
                vidx.at[pl.ds(0, ids_per_unit)],
                rows.at[p, pl.ds(0, ids_per_unit)],
                sem.at[p]).wait()

        def compute(un, p):
            def body(g, c2):
                wv16 = w_v[pl.ds(un * UN + g * 16, 16)]
                if negate:
                    wv16 = -wv16
                for j in range(16):
                    i = g * 16 + j
                    wv = jnp.full((16,), wv16[j], jnp.float32)
                    m8 = jnp.maximum(rows[p, i, pl.ds(0, 64)],
                                     rows[p, UN + i, pl.ds(0, 64)])
                    if rows_per == 3:
                        m8 = jnp.maximum(m8, rows[p, 2 * UN + i, pl.ds(0, 64)])
                    f_lo, f_hi = _unpack_idx(m8, offs_lo, offs_hi)
                    plsc.addupdate_scatter(hist, [f_lo], wv)
                    plsc.addupdate_scatter(hist, [f_hi], wv)
                return c2

            lax.fori_loop(0, UN // 16, body, 0)

        def super_body(k, carry):
            stage(k)
            fire(0, 0)
            fire(1, 1)
            for un in range(NUS):
                p = un & 1
                wait(p)
                compute(un, p)
                if un + 2 < NUS:
                    fire(un + 2, p)
            return carry

        lax.fori_loop(0, n_super, super_body, 0)

    simplex_pass([ev0, ev1], ew, NSE, 2, False)
    simplex_pass([tv0, tv1, tv2], tw, NST, 3, True)

    pltpu.sync_copy(hist, out.at[pl.ds(wid * DH, DH)])


def _make_sc_hist():
    mesh = plsc.VectorSubcoreMesh(core_axis_name="c", subcore_axis_name="s")
    return functools.partial(
        pl.kernel,
        mesh=mesh,
        compiler_params=pltpu.CompilerParams(
            needs_layout_passes=False, use_tc_tiling_on_sc=False),
        out_type=jax.ShapeDtypeStruct((NW * DH,), jnp.float32),
        scratch_types=[
            pltpu.VMEM((DH,), jnp.float32),
            pltpu.VMEM((SB * 3,), jnp.int32),
            pltpu.VMEM((2, UN * 3, 128), jnp.int8),
            pltpu.VMEM((SB,), jnp.float32),
            pltpu.SemaphoreType.DMA((2,)),
        ],
    )(_sc_hist_body)


_sc_hist = _make_sc_hist()


def kernel(v_coords, v_weights, edge_verts, edge_weights, tri_verts,
           tri_weights, dirs):
    vc = jnp.pad(v_coords, ((0, NVP - N_V), (0, 0)))
    vwp = jnp.pad(v_weights, (0, NVP - N_V))
    ev0 = jnp.pad(edge_verts[:, 0].astype(jnp.int32), (0, NEP - N_E))
    ev1 = jnp.pad(edge_verts[:, 1].astype(jnp.int32), (0, NEP - N_E))
    ewp = jnp.pad(edge_weights, (0, NEP - N_E))
    tv0 = jnp.pad(tri_verts[:, 0].astype(jnp.int32), (0, NTP - N_T))
    tv1 = jnp.pad(tri_verts[:, 1].astype(jnp.int32), (0, NTP - N_T))
    tv2 = jnp.pad(tri_verts[:, 2].astype(jnp.int32), (0, NTP - N_T))
    twp = jnp.pad(tri_weights, (0, NTP - N_T))

    dirs_t = dirs.T
    dmat = jnp.zeros((3, 128), jnp.float32)
    cols = jnp.arange(16) * 4
    dmat = dmat.at[:, cols].set(dirs_t[:, :16])
    dmat = dmat.at[:, cols + 1].set(dirs_t[:, 16:])

    maxsq = pl.pallas_call(
        _maxsq_body,
        grid=(NGRID,),
        in_specs=[pl.BlockSpec((VBLK, 3), lambda i: (i, 0))],
        out_specs=pl.BlockSpec(memory_space=pltpu.SMEM),
        out_shape=jax.ShapeDtypeStruct((1, 1), jnp.float32),
    )(vc)

    vidx = pl.pallas_call(
        _quant_body,
        grid=(NGRID,),
        in_specs=[
            pl.BlockSpec(memory_space=pltpu.SMEM),
            pl.BlockSpec((VBLK, 3), lambda i: (i, 0)),
            pl.BlockSpec((3, 128), lambda i: (0, 0)),
        ],
        out_specs=pl.BlockSpec((VBLK, 128), lambda i: (i, 0)),
        out_shape=jax.ShapeDtypeStruct((NVP, 128), jnp.int8),
    )(maxsq, vc, dmat)

    hists = _sc_hist(vidx, vwp, ev0, ev1, ewp, tv0, tv1, tv2, twp)

    out = pl.pallas_call(
        _fin_body,
        in_specs=[pl.BlockSpec((NW * D, H), lambda: (0, 0))],
        out_specs=pl.BlockSpec((D, H), lambda: (0, 0)),
        out_shape=jax.ShapeDtypeStruct((D, H), jnp.float32),
    )(hists.reshape(NW * D, H))

    return out


# speedup vs baseline: 28.6935x; 1.0307x over previous
import functools

import jax
import jax.numpy as jnp
from jax import lax
from jax.experimental import pallas as pl
from jax.experimental.pallas import tpu as pltpu
from jax.experimental.pallas import tpu_sc as plsc

D = 32
H = 256
DH = D * H

N_V = 100000
N_E = 500000
N_T = 500000

NW = 32
NC = 2

NVP = 102400
NEP = 524288
NTP = 524288

VPT = NVP // NW
EPT = NEP // NW
TPT = NTP // NW

SB = 2048
UN = 512
NUS = SB // UN
NSE = EPT // SB
NST = TPT // SB

VB = 128
NVB = VPT // VB

VBLK = 2048
NGRID = NVP // VBLK


def _maxsq_body(x_ref, o_ref):
    i = pl.program_id(0)
    c = x_ref[...]
    s = jnp.sum(c * c, axis=1)
    m = jnp.max(s)

    @pl.when(i == 0)
    def _():
        o_ref[0, 0] = m

    @pl.when(i > 0)
    def _():
        o_ref[0, 0] = jnp.maximum(o_ref[0, 0], m)


def _quant_body(ms_ref, x_ref, d_ref, o_ref):
    m = jnp.sqrt(ms_ref[0, 0])
    c = x_ref[...]
    dm = d_ref[...]
    h = (c[:, 0:1] * dm[0:1, :]
         + c[:, 1:2] * dm[1:2, :]
         + c[:, 2:3] * dm[2:3, :])
    idx = jnp.ceil((jnp.float32(H - 1) * (m + h)) / (jnp.float32(2.0) * m))
    idx = jnp.clip(idx, 0.0, jnp.float32(H - 1))
    col = lax.broadcasted_iota(jnp.int32, (VBLK, 128), 1)
    bins = jnp.where(col % 4 == 0, idx.astype(jnp.int32), 0)
    out = bins + jnp.where(col % 4 == 1, col // 4, 0)
    o_ref[...] = out.astype(jnp.uint8)


def _fin_body(h_ref, o_ref):
    x = h_ref[...]
    j = lax.broadcasted_iota(jnp.int32, (D, NW * D), 1)
    dd = lax.broadcasted_iota(jnp.int32, (D, NW * D), 0)
    sel = ((j & (D - 1)) == dd).astype(jnp.float32)
    acc = jnp.dot(sel, x, preferred_element_type=jnp.float32)
    s_i = lax.broadcasted_iota(jnp.int32, (H, H), 0)
    t_i = lax.broadcasted_iota(jnp.int32, (H, H), 1)
    tri = (s_i <= t_i).astype(jnp.float32)
    o_ref[...] = jnp.dot(acc, tri, preferred_element_type=jnp.float32)


def _sc_hist_body(vidx, vw, ev0, ev1, ew, tv0, tv1, tv2, tw, out,
                  hist, idx_v, rows, w_v, sem):
    wid = lax.axis_index("s") * NC + lax.axis_index("c")

    zeros16 = jnp.zeros((16,), jnp.float32)

    def zi(i, _):
        hist[pl.ds(i * 16, 16)] = zeros16
        return 0

    lax.fori_loop(0, DH // 16, zi, 0)

    vbase = wid * VPT

    def vblk(b, carry):
        base = vbase + b * VB
        pltpu.sync_copy(vidx.at[pl.ds(base, VB)], rows.at[0, pl.ds(0, VB)])
        pltpu.sync_copy(vw.at[pl.ds(base, VB)], w_v.at[pl.ds(0, VB)])

        def vbody(g, c2):
            wv16 = w_v[pl.ds(g * 16, 16)]
            for j in range(16):
                i = g * 16 + j
                wv = jnp.full((16,), wv16[j], jnp.float32)
                f_lo = plsc.bitcast(rows[0, i, pl.ds(0, 64)], jnp.int32)
                f_hi = plsc.bitcast(rows[0, i, pl.ds(64, 64)], jnp.int32)
                plsc.addupdate_scatter(hist, [f_lo], wv)
                plsc.addupdate_scatter(hist, [f_hi], wv)
            return c2

        return lax.fori_loop(0, VB // 16, vbody, carry)

    lax.fori_loop(0, NVB, vblk, 0)

    def simplex_pass(cols, w_hbm, n_super, rows_per, negate):
        ids_per_unit = UN * rows_per
        base = wid * (EPT if rows_per == 2 else TPT)

        def stage(k):
            for r in range(rows_per):
                pltpu.sync_copy(cols[r].at[pl.ds(base + k * SB, SB)],
                                idx_v.at[pl.ds(r * SB, SB)])
            pltpu.sync_copy(w_hbm.at[pl.ds(base + k * SB, SB)],
                            w_v.at[pl.ds(0, SB)])

        def fire(un, p):
            for r in range(rows_per):
                for s in range(UN // 128):
                    pltpu.async_copy(
                        vidx.at[idx_v.at[pl.ds(r * SB + un * UN + s * 128,
                                               128)]],
                        rows.at[p, pl.ds(r * UN + s * 128, 128)],
                        sem.at[p])

        def wait(p):
            pltpu.make_async_copy(
                vidx.at[pl.ds(0, ids_per_unit)],
                rows.at[p, pl.ds(0, ids_per_unit)],
                sem.at[p]).wait()

        def compute(un, p):
            def body(g, c2):
                wv16 = w_v[pl.ds(un * UN + g * 16, 16)]
                if negate:
                    wv16 = -wv16
                for j in range(16):
                    i = g * 16 + j
                    wv = jnp.full((16,), wv16[j], jnp.float32)

                    def word(r, h):
                        return plsc.bitcast(rows[p, r, pl.ds(h * 64, 64)],
                                            jnp.int32)

                    ma = jnp.maximum(word(i, 0), word(UN + i, 0))
                    mb = jnp.maximum(word(i, 1), word(UN + i, 1))
                    if rows_per == 3:
                        ma = jnp.maximum(ma, word(2 * UN + i, 0))
                        mb = jnp.maximum(mb, word(2 * UN + i, 1))
                    plsc.addupdate_scatter(hist, [ma], wv)
                    plsc.addupdate_scatter(hist, [mb], wv)
                return c2

            lax.fori_loop(0, UN // 16, body, 0)

        def super_body(k, carry):
            stage(k)
            fire(0, 0)
            fire(1, 1)
            for un in range(NUS):
                p = un & 1
                wait(p)
                compute(un, p)
                if un + 2 < NUS:
                    fire(un + 2, p)
            return carry

        lax.fori_loop(0, n_super, super_body, 0)

    simplex_pass([ev0, ev1], ew, NSE, 2, False)
    simplex_pass([tv0, tv1, tv2], tw, NST, 3, True)

    pltpu.sync_copy(hist, out.at[pl.ds(wid * DH, DH)])


def _make_sc_hist():
    mesh = plsc.VectorSubcoreMesh(core_axis_name="c", subcore_axis_name="s")
    return functools.partial(
        pl.kernel,
        mesh=mesh,
        compiler_params=pltpu.CompilerParams(
            needs_layout_passes=False, use_tc_tiling_on_sc=False),
        out_type=jax.ShapeDtypeStruct((NW * DH,), jnp.float32),
        scratch_types=[
            pltpu.VMEM((DH,), jnp.float32),
            pltpu.VMEM((SB * 3,), jnp.int32),
            pltpu.VMEM((2, UN * 3, 128), jnp.uint8),
            pltpu.VMEM((SB,), jnp.float32),
            pltpu.SemaphoreType.DMA((2,)),
        ],
    )(_sc_hist_body)


_sc_hist = _make_sc_hist()


def kernel(v_coords, v_weights, edge_verts, edge_weights, tri_verts,
           tri_weights, dirs):
    vc = jnp.pad(v_coords, ((0, NVP - N_V), (0, 0)))
    vwp = jnp.pad(v_weights, (0, NVP - N_V))
    ev0 = jnp.pad(edge_verts[:, 0].astype(jnp.int32), (0, NEP - N_E))
    ev1 = jnp.pad(edge_verts[:, 1].astype(jnp.int32), (0, NEP - N_E))
    ewp = jnp.pad(edge_weights, (0, NEP - N_E))
    tv0 = jnp.pad(tri_verts[:, 0].astype(jnp.int32), (0, NTP - N_T))
    tv1 = jnp.pad(tri_verts[:, 1].astype(jnp.int32), (0, NTP - N_T))
    tv2 = jnp.pad(tri_verts[:, 2].astype(jnp.int32), (0, NTP - N_T))
    twp = jnp.pad(tri_weights, (0, NTP - N_T))

    dirs_t = dirs.T
    dmat = jnp.zeros((3, 128), jnp.float32)
    dmat = dmat.at[:, jnp.arange(D) * 4].set(dirs_t)

    maxsq = pl.pallas_call(
        _maxsq_body,
        grid=(NGRID,),
        in_specs=[pl.BlockSpec((VBLK, 3), lambda i: (i, 0))],
        out_specs=pl.BlockSpec(memory_space=pltpu.SMEM),
        out_shape=jax.ShapeDtypeStruct((1, 1), jnp.float32),
    )(vc)

    vidx = pl.pallas_call(
        _quant_body,
        grid=(NGRID,),
        in_specs=[
            pl.BlockSpec(memory_space=pltpu.SMEM),
            pl.BlockSpec((VBLK, 3), lambda i: (i, 0)),
            pl.BlockSpec((3, 128), lambda i: (0, 0)),
        ],
        out_specs=pl.BlockSpec((VBLK, 128), lambda i: (i, 0)),
        out_shape=jax.ShapeDtypeStruct((NVP, 128), jnp.uint8),
    )(maxsq, vc, dmat)

    hists = _sc_hist(vidx, vwp, ev0, ev1, ewp, tv0, tv1, tv2, twp)

    out = pl.pallas_call(
        _fin_body,
        in_specs=[pl.BlockSpec((NW * D, H), lambda: (0, 0))],
        out_specs=pl.BlockSpec((D, H), lambda: (0, 0)),
        out_shape=jax.ShapeDtypeStruct((D, H), jnp.float32),
    )(hists.reshape(NW * D, H))

    return out


# speedup vs baseline: 374.7623x; 13.0609x over previous
import functools

import jax
import jax.numpy as jnp
from jax import lax
from jax.experimental import pallas as pl
from jax.experimental.pallas import tpu as pltpu
from jax.experimental.pallas import tpu_sc as plsc

D = 32
H = 256
NSUB = 16
SHW = NSUB * H

N_V = 100000
N_E = 500000
N_T = 500000

NW = 32
NC = 2

NVP = 102400
NEP = 524288
NTP = 524288

SB = 1024
NSV = NVP // SB
NSE = NEP // SB
NST = NTP // SB

VBLK = 2048
NGRID = NVP // VBLK


def _maxsq_body(x_ref, y_ref, z_ref, o_ref):
    i = pl.program_id(0)
    x = x_ref[...]
    y = y_ref[...]
    z = z_ref[...]
    m = jnp.max(x * x + y * y + z * z)

    @pl.when(i == 0)
    def _():
        o_ref[0, 0] = m

    @pl.when(i > 0)
    def _():
        o_ref[0, 0] = jnp.maximum(o_ref[0, 0], m)


def _quant_body(ms_ref, x_ref, y_ref, z_ref, d_ref, o_ref):
    m = jnp.sqrt(ms_ref[0, 0])
    dm = d_ref[...]
    h = (dm[:, 0:1] * x_ref[...]
         + dm[:, 1:2] * y_ref[...]
         + dm[:, 2:3] * z_ref[...])
    idx = jnp.ceil((jnp.float32(H - 1) * (m + h)) / (jnp.float32(2.0) * m))
    idx = jnp.clip(idx, 0.0, jnp.float32(H - 1))
    o_ref[...] = idx.astype(jnp.int32)


def _fin_body(h_ref, o_ref):
    x = h_ref[...]
    j = lax.broadcasted_iota(jnp.int32, (D, NW * NSUB), 1)
    dd = lax.broadcasted_iota(jnp.int32, (D, NW * NSUB), 0)
    sel = ((j // NSUB) == dd).astype(jnp.float32)
    acc = jnp.dot(sel, x, preferred_element_type=jnp.float32)
    s_i = lax.broadcasted_iota(jnp.int32, (H, H), 0)
    t_i = lax.broadcasted_iota(jnp.int32, (H, H), 1)
    tri = (s_i <= t_i).astype(jnp.float32)
    o_ref[...] = jnp.dot(acc, tri, preferred_element_type=jnp.float32)


def _sc_hist_body(tbl, vw, ev0, ev1, ew, tv0, tv1, tv2, tw, out,
                  binv, subh, ids, wbuf, sem):
    wid = lax.axis_index("s") * NC + lax.axis_index("c")

    zeros16 = jnp.zeros((16,), jnp.float32)

    def zi(i, _):
        subh[pl.ds(i * 16, 16)] = zeros16
        return 0

    lax.fori_loop(0, SHW // 16, zi, 0)

    pltpu.sync_copy(tbl.at[pl.ds(wid * NVP, NVP)], binv)

    laneoff = lax.iota(jnp.int32, 16) * H

    def vsb(k, carry):
        pltpu.sync_copy(vw.at[pl.ds(k * SB, SB)], wbuf.at[0])

        def vbody(g, c2):
            b16 = binv[pl.ds(k * SB + g * 16, 16)]
            w16 = wbuf[0, pl.ds(g * 16, 16)]
            plsc.addupdate_scatter(subh, [b16 + laneoff], w16)
            return c2

        return lax.fori_loop(0, SB // 16, vbody, carry)

    lax.fori_loop(0, NSV, vsb, 0)

    def simplex_pass(cols, w_hbm, n_super, negate):
        rows_per = len(cols)

        def stage(k, p):
            for r in range(rows_per):
                pltpu.async_copy(cols[r].at[pl.ds(k * SB, SB)],
                                 ids.at[p, pl.ds(r * SB, SB)],
                                 sem.at[p])
            pltpu.async_copy(w_hbm.at[pl.ds(k * SB, SB)], wbuf.at[p],
                             sem.at[p])

        def wait(p):
            pltpu.make_async_copy(
                cols[0].at[pl.ds(0, rows_per * SB)],
                ids.at[p, pl.ds(0, rows_per * SB)],
                sem.at[p]).wait()
            pltpu.make_async_copy(
                w_hbm.at[pl.ds(0, SB)], wbuf.at[p], sem.at[p]).wait()

        def compute(p):
            def body(g, c2):
                i0 = ids[p, pl.ds(g * 16, 16)]
                i1 = ids[p, pl.ds(SB + g * 16, 16)]
                b = jnp.maximum(plsc.load_gather(binv, [i0]),
                                plsc.load_gather(binv, [i1]))
                if rows_per == 3:
                    i2 = ids[p, pl.ds(2 * SB + g * 16, 16)]
                    b = jnp.maximum(b, plsc.load_gather(binv, [i2]))
                w16 = wbuf[p, pl.ds(g * 16, 16)]
                if negate:
                    w16 = -w16
                plsc.addupdate_scatter(subh, [b + laneoff], w16)
                return c2

            lax.fori_loop(0, SB // 16, body, 0)

        stage(0, 0)

        def pair(j, carry):
            stage(2 * j + 1, 1)
            wait(0)
            compute(0)

            @pl.when(j + 1 < n_super // 2)
            def _():
                stage(2 * j + 2, 0)

            wait(1)
            compute(1)
            return carry

        lax.fori_loop(0, n_super // 2, pair, 0)

    simplex_pass([ev0, ev1], ew, NSE, False)
    simplex_pass([tv0, tv1, tv2], tw, NST, True)

    pltpu.sync_copy(subh, out.at[pl.ds(wid * SHW, SHW)])


def _make_sc_hist():
    mesh = plsc.VectorSubcoreMesh(core_axis_name="c", subcore_axis_name="s")
    return functools.partial(
        pl.kernel,
        mesh=mesh,
        compiler_params=pltpu.CompilerParams(
            needs_layout_passes=False, use_tc_tiling_on_sc=False),
        out_type=jax.ShapeDtypeStruct((NW * SHW,), jnp.float32),
        scratch_types=[
            pltpu.VMEM((NVP,), jnp.int32),
            pltpu.VMEM((SHW,), jnp.float32),
            pltpu.VMEM((2, 3 * SB), jnp.int32),
            pltpu.VMEM((2, SB), jnp.float32),
            pltpu.SemaphoreType.DMA((2,)),
        ],
    )(_sc_hist_body)


_sc_hist = _make_sc_hist()


def kernel(v_coords, v_weights, edge_verts, edge_weights, tri_verts,
           tri_weights, dirs):
    cx = jnp.pad(v_coords[:, 0], (0, NVP - N_V)).reshape(1, NVP)
    cy = jnp.pad(v_coords[:, 1], (0, NVP - N_V)).reshape(1, NVP)
    cz = jnp.pad(v_coords[:, 2], (0, NVP - N_V)).reshape(1, NVP)
    vwp = jnp.pad(v_weights, (0, NVP - N_V))
    ev0 = jnp.pad(edge_verts[:, 0].astype(jnp.int32), (0, NEP - N_E))
    ev1 = jnp.pad(edge_verts[:, 1].astype(jnp.int32), (0, NEP - N_E))
    ewp = jnp.pad(edge_weights, (0, NEP - N_E))
    tv0 = jnp.pad(tri_verts[:, 0].astype(jnp.int32), (0, NTP - N_T))
    tv1 = jnp.pad(tri_verts[:, 1].astype(jnp.int32), (0, NTP - N_T))
    tv2 = jnp.pad(tri_verts[:, 2].astype(jnp.int32), (0, NTP - N_T))
    twp = jnp.pad(tri_weights, (0, NTP - N_T))

    maxsq = pl.pallas_call(
        _maxsq_body,
        grid=(NGRID,),
        in_specs=[pl.BlockSpec((1, VBLK), lambda i: (0, i))] * 3,
        out_specs=pl.BlockSpec(memory_space=pltpu.SMEM),
        out_shape=jax.ShapeDtypeStruct((1, 1), jnp.float32),
    )(cx, cy, cz)

    tbl = pl.pallas_call(
        _quant_body,
        grid=(NGRID,),
        in_specs=[
            pl.BlockSpec(memory_space=pltpu.SMEM),
            pl.BlockSpec((1, VBLK), lambda i: (0, i)),
            pl.BlockSpec((1, VBLK), lambda i: (0, i)),
            pl.BlockSpec((1, VBLK), lambda i: (0, i)),
            pl.BlockSpec((D, 3), lambda i: (0, 0)),
        ],
        out_specs=pl.BlockSpec((D, VBLK), lambda i: (0, i)),
        out_shape=jax.ShapeDtypeStruct((D, NVP), jnp.int32),
    )(maxsq, cx, cy, cz, dirs)

    hists = _sc_hist(tbl.reshape(-1), vwp, ev0, ev1, ewp,
                     tv0, tv1, tv2, twp)

    out = pl.pallas_call(
        _fin_body,
        in_specs=[pl.BlockSpec((NW * NSUB, H), lambda: (0, 0))],
        out_specs=pl.BlockSpec((D, H), lambda: (0, 0)),
        out_shape=jax.ShapeDtypeStruct((D, H), jnp.float32),
    )(hists.reshape(NW * NSUB, H))

    return out
